# Initial kernel scaffold; baseline (speedup 1.0000x reference)
import jax
import jax.numpy as jnp
from jax.experimental import pallas as pl


def kernel(h, edge_index, W1a, b1a, W1b, b1b, W4a, b4a, W4b, b4b):
    raise NotImplementedError("write your pallas kernel here")


# speedup vs baseline: 4.6235x; 4.6235x over previous
import functools

import jax
import jax.numpy as jnp
from jax import lax
from jax.experimental import pallas as pl
from jax.experimental.pallas import tpu as pltpu
from jax.experimental.pallas import tpu_sc as plsc

N = 10000
E = 320000
D = 128

NC = 2
NS = 16
NW = NC * NS
EPT = E // NW
C = 80
NCHUNK = EPT // C
RPT = 624
TAIL = N - NS * RPT


def _segsum_body(x_hbm, src_hbm, dst_hbm, z_hbm, out_hbm,
                 sidx, didx, rows, acc, sem):
    c = lax.axis_index("c")
    s = lax.axis_index("s")
    wid = c * NS + s
    rbase = s * RPT
    pltpu.sync_copy(z_hbm.at[pl.ds(rbase, RPT)], acc.at[pl.ds(rbase, RPT)])

    @pl.when(s == NS - 1)
    def _():
        pltpu.sync_copy(z_hbm.at[pl.ds(NS * RPT, TAIL)],
                        acc.at[pl.ds(NS * RPT, TAIL)])

    plsc.subcore_barrier()

    ebase = wid * EPT

    def chunk(i, carry):
        off = ebase + i * C
        pltpu.sync_copy(src_hbm.at[pl.ds(off, C)], sidx)
        pltpu.sync_copy(dst_hbm.at[pl.ds(off, C)], didx)
        pltpu.async_copy(x_hbm.at[sidx], rows, sem).wait()
        pltpu.sync_copy(rows, acc.at[didx], add=True)
        return carry

    lax.fori_loop(0, NCHUNK, chunk, 0)
    plsc.subcore_barrier()
    pltpu.sync_copy(acc.at[pl.ds(rbase, RPT)],
                    out_hbm.at[c].at[pl.ds(rbase, RPT)])

    @pl.when(s == NS - 1)
    def _():
        pltpu.sync_copy(acc.at[pl.ds(NS * RPT, TAIL)],
                        out_hbm.at[c].at[pl.ds(NS * RPT, TAIL)])


@functools.cache
def _get_segsum():
    return pl.kernel(
        _segsum_body,
        out_type=jax.ShapeDtypeStruct((NC, N, D), jnp.float32),
        mesh=plsc.VectorSubcoreMesh(core_axis_name="c", subcore_axis_name="s",
                                    num_cores=NC, num_subcores=NS),
        scratch_types=[
            pltpu.VMEM((C,), jnp.int32),
            pltpu.VMEM((C,), jnp.int32),
            pltpu.VMEM((C, D), jnp.float32),
            pltpu.VMEM_SHARED((N, D), jnp.float32),
            pltpu.SemaphoreType.DMA,
        ],
    )


BLK = 1000


def _mm(x, w):
    return lax.dot_general(x, w, (((1,), (0,)), ((), ())),
                           preferred_element_type=jnp.float32,
                           precision=lax.Precision.HIGHEST)


def _mlp_stats_body(h_ref, a0_ref, a1_ref, wa_ref, ba_ref, wb_ref, bb_ref,
                    l1_ref, st_ref):
    u = h_ref[...] + a0_ref[...] + a1_ref[...]
    t = jnp.maximum(_mm(u, wa_ref[...]) + ba_ref[...], 0.0)
    l1 = _mm(t, wb_ref[...]) + bb_ref[...]
    l1_ref[...] = l1

    @pl.when(pl.program_id(0) == 0)
    def _():
        st_ref[...] = jnp.zeros_like(st_ref)

    colsum = jnp.sum(l1, axis=0, keepdims=True)
    sumsq = jnp.broadcast_to(jnp.sum(l1 * l1), (1, D))
    st_ref[...] += jnp.concatenate(
        [colsum, sumsq, jnp.zeros((6, D), jnp.float32)], axis=0)


_mlp_stats = pl.pallas_call(
    _mlp_stats_body,
    grid=(N // BLK,),
    in_specs=[
        pl.BlockSpec((BLK, D), lambda i: (i, 0)),
        pl.BlockSpec((BLK, D), lambda i: (i, 0)),
        pl.BlockSpec((BLK, D), lambda i: (i, 0)),
        pl.BlockSpec((D, D), lambda i: (0, 0)),
        pl.BlockSpec((1, D), lambda i: (0, 0)),
        pl.BlockSpec((D, D), lambda i: (0, 0)),
        pl.BlockSpec((1, D), lambda i: (0, 0)),
    ],
    out_specs=[
        pl.BlockSpec((BLK, D), lambda i: (i, 0)),
        pl.BlockSpec((8, D), lambda i: (0, 0)),
    ],
    out_shape=[
        jax.ShapeDtypeStruct((N, D), jnp.float32),
        jax.ShapeDtypeStruct((8, D), jnp.float32),
    ],
)


def _norm_relu_body(l1_ref, musc_ref, inv_ref, out_ref):
    out_ref[...] = jnp.maximum(l1_ref[...] * inv_ref[...] - musc_ref[...], 0.0)


_norm_relu = pl.pallas_call(
    _norm_relu_body,
    grid=(N // BLK,),
    in_specs=[
        pl.BlockSpec((BLK, D), lambda i: (i, 0)),
        pl.BlockSpec((1, D), lambda i: (0, 0)),
        pl.BlockSpec((1, D), lambda i: (0, 0)),
    ],
    out_specs=pl.BlockSpec((BLK, D), lambda i: (i, 0)),
    out_shape=jax.ShapeDtypeStruct((N, D), jnp.float32),
)


def _mlp_body(h_ref, a0_ref, a1_ref, wa_ref, ba_ref, wb_ref, bb_ref, out_ref):
    u = h_ref[...] + a0_ref[...] + a1_ref[...]
    t = jnp.maximum(_mm(u, wa_ref[...]) + ba_ref[...], 0.0)
    out_ref[...] = _mm(t, wb_ref[...]) + bb_ref[...]


_mlp = pl.pallas_call(
    _mlp_body,
    grid=(N // BLK,),
    in_specs=[
        pl.BlockSpec((BLK, D), lambda i: (i, 0)),
        pl.BlockSpec((BLK, D), lambda i: (i, 0)),
        pl.BlockSpec((BLK, D), lambda i: (i, 0)),
        pl.BlockSpec((D, D), lambda i: (0, 0)),
        pl.BlockSpec((1, D), lambda i: (0, 0)),
        pl.BlockSpec((D, D), lambda i: (0, 0)),
        pl.BlockSpec((1, D), lambda i: (0, 0)),
    ],
    out_specs=pl.BlockSpec((BLK, D), lambda i: (i, 0)),
    out_shape=jax.ShapeDtypeStruct((N, D), jnp.float32),
)


def kernel(h, edge_index, W1a, b1a, W1b, b1b, W4a, b4a, W4b, b4b):
    src = edge_index[0].astype(jnp.int32)
    dst = edge_index[1].astype(jnp.int32)
    zeros = jnp.zeros((N, D), jnp.float32)

    _segsum = _get_segsum()
    agg1 = _segsum(h, src, dst, zeros)
    l1, st = _mlp_stats(h, agg1[0], agg1[1],
                        W1a, b1a.reshape(1, D), W1b, b1b.reshape(1, D))
    mu = st[0] / N
    msq = st[1, 0] / N
    inv = lax.rsqrt(1e-6 + msq - jnp.sum(mu * mu))
    l1n = _norm_relu(l1, (mu * inv).reshape(1, D),
                     jnp.full((1, D), inv, jnp.float32))

    agg2 = _segsum(l1n, src, dst, zeros)
    l5 = _mlp(l1n, agg2[0], agg2[1],
              W4a, b4a.reshape(1, D), W4b, b4b.reshape(1, D))
    return (l5, l1n)


# speedup vs baseline: 9.4158x; 2.0365x over previous
import functools

import jax
import jax.numpy as jnp
from jax import lax
from jax.experimental import pallas as pl
from jax.experimental.pallas import tpu as pltpu
from jax.experimental.pallas import tpu_sc as plsc

N = 10000
E = 320000
D = 128

NC = 2
NS = 16
NW = NC * NS
EPT = E // NW
C = 80
NCHUNK = EPT // C
RPT = 624
TAIL = N - NS * RPT


def _segsum_body(x_hbm, src_hbm, dst_hbm, z_hbm, out_hbm,
                 sidx, didx, rows_a, rows_b, acc, sem_i, sem_a, sem_b):
    c = lax.axis_index("c")
    s = lax.axis_index("s")
    wid = c * NS + s
    rbase = s * RPT

    ci = pltpu.async_copy(src_hbm.at[wid], sidx, sem_i)
    cd = pltpu.async_copy(dst_hbm.at[wid], didx, sem_i)
    pltpu.sync_copy(z_hbm.at[pl.ds(rbase, RPT)], acc.at[pl.ds(rbase, RPT)])

    @pl.when(s == NS - 1)
    def _():
        pltpu.sync_copy(z_hbm.at[pl.ds(NS * RPT, TAIL)],
                        acc.at[pl.ds(NS * RPT, TAIL)])

    ci.wait()
    cd.wait()
    plsc.subcore_barrier()

    def gather(i, rbuf, sem):
        pltpu.async_copy(x_hbm.at[sidx.at[pl.ds(i * C, C)]], rbuf, sem)

    def gwait(rbuf, sem):
        pltpu.make_async_copy(x_hbm.at[sidx.at[pl.ds(0, C)]], rbuf, sem).wait()

    def scat(i, rbuf):
        pltpu.sync_copy(rbuf, acc.at[didx.at[i]], add=True)

    gather(0, rows_a, sem_a)

    def pair(b, carry):
        i = 2 * b
        gather(i + 1, rows_b, sem_b)
        gwait(rows_a, sem_a)
        scat(i, rows_a)

        gather(i + 2, rows_a, sem_a)
        gwait(rows_b, sem_b)
        scat(i + 1, rows_b)
        return carry

    lax.fori_loop(0, NCHUNK // 2, pair, 0)
    gwait(rows_a, sem_a)
    scat(NCHUNK - 1, rows_a)
    plsc.subcore_barrier()
    pltpu.sync_copy(acc.at[pl.ds(rbase, RPT)],
                    out_hbm.at[c].at[pl.ds(rbase, RPT)])

    @pl.when(s == NS - 1)
    def _():
        pltpu.sync_copy(acc.at[pl.ds(NS * RPT, TAIL)],
                        out_hbm.at[c].at[pl.ds(NS * RPT, TAIL)])


@functools.cache
def _get_segsum():
    return pl.kernel(
        _segsum_body,
        out_type=jax.ShapeDtypeStruct((NC, N, D), jnp.float32),
        mesh=plsc.VectorSubcoreMesh(core_axis_name="c", subcore_axis_name="s",
                                    num_cores=NC, num_subcores=NS),
        scratch_types=[
            pltpu.VMEM((EPT,), jnp.int32),
            pltpu.VMEM((NCHUNK, C), jnp.int32),
            pltpu.VMEM((C, D), jnp.float32),
            pltpu.VMEM((C, D), jnp.float32),
            pltpu.VMEM_SHARED((N, D), jnp.float32),
            pltpu.SemaphoreType.DMA,
            pltpu.SemaphoreType.DMA,
            pltpu.SemaphoreType.DMA,
        ],
    )


BLK = 1000


def _mm(x, w):
    return lax.dot_general(x, w, (((1,), (0,)), ((), ())),
                           preferred_element_type=jnp.float32,
                           precision=lax.Precision.HIGHEST)


def _mlp_stats_body(h_ref, a0_ref, a1_ref, wa_ref, ba_ref, wb_ref, bb_ref,
                    l1_ref, st_ref):
    u = h_ref[...] + a0_ref[...] + a1_ref[...]
    t = jnp.maximum(_mm(u, wa_ref[...]) + ba_ref[...], 0.0)
    l1 = _mm(t, wb_ref[...]) + bb_ref[...]
    l1_ref[...] = l1

    @pl.when(pl.program_id(0) == 0)
    def _():
        st_ref[...] = jnp.zeros_like(st_ref)

    colsum = jnp.sum(l1, axis=0, keepdims=True)
    sumsq = jnp.broadcast_to(jnp.sum(l1 * l1), (1, D))
    st_ref[...] += jnp.concatenate(
        [colsum, sumsq, jnp.zeros((6, D), jnp.float32)], axis=0)


_mlp_stats = pl.pallas_call(
    _mlp_stats_body,
    grid=(N // BLK,),
    in_specs=[
        pl.BlockSpec((BLK, D), lambda i: (i, 0)),
        pl.BlockSpec((BLK, D), lambda i: (i, 0)),
        pl.BlockSpec((BLK, D), lambda i: (i, 0)),
        pl.BlockSpec((D, D), lambda i: (0, 0)),
        pl.BlockSpec((1, D), lambda i: (0, 0)),
        pl.BlockSpec((D, D), lambda i: (0, 0)),
        pl.BlockSpec((1, D), lambda i: (0, 0)),
    ],
    out_specs=[
        pl.BlockSpec((BLK, D), lambda i: (i, 0)),
        pl.BlockSpec((8, D), lambda i: (0, 0)),
    ],
    out_shape=[
        jax.ShapeDtypeStruct((N, D), jnp.float32),
        jax.ShapeDtypeStruct((8, D), jnp.float32),
    ],
)


def _norm_relu_body(l1_ref, musc_ref, inv_ref, out_ref):
    out_ref[...] = jnp.maximum(l1_ref[...] * inv_ref[...] - musc_ref[...], 0.0)


_norm_relu = pl.pallas_call(
    _norm_relu_body,
    grid=(N // BLK,),
    in_specs=[
        pl.BlockSpec((BLK, D), lambda i: (i, 0)),
        pl.BlockSpec((1, D), lambda i: (0, 0)),
        pl.BlockSpec((1, D), lambda i: (0, 0)),
    ],
    out_specs=pl.BlockSpec((BLK, D), lambda i: (i, 0)),
    out_shape=jax.ShapeDtypeStruct((N, D), jnp.float32),
)


def _mlp_body(h_ref, a0_ref, a1_ref, wa_ref, ba_ref, wb_ref, bb_ref, out_ref):
    u = h_ref[...] + a0_ref[...] + a1_ref[...]
    t = jnp.maximum(_mm(u, wa_ref[...]) + ba_ref[...], 0.0)
    out_ref[...] = _mm(t, wb_ref[...]) + bb_ref[...]


_mlp = pl.pallas_call(
    _mlp_body,
    grid=(N // BLK,),
    in_specs=[
        pl.BlockSpec((BLK, D), lambda i: (i, 0)),
        pl.BlockSpec((BLK, D), lambda i: (i, 0)),
        pl.BlockSpec((BLK, D), lambda i: (i, 0)),
        pl.BlockSpec((D, D), lambda i: (0, 0)),
        pl.BlockSpec((1, D), lambda i: (0, 0)),
        pl.BlockSpec((D, D), lambda i: (0, 0)),
        pl.BlockSpec((1, D), lambda i: (0, 0)),
    ],
    out_specs=pl.BlockSpec((BLK, D), lambda i: (i, 0)),
    out_shape=jax.ShapeDtypeStruct((N, D), jnp.float32),
)


def kernel(h, edge_index, W1a, b1a, W1b, b1b, W4a, b4a, W4b, b4b):
    src = edge_index[0].astype(jnp.int32).reshape(NW, EPT)
    dst = edge_index[1].astype(jnp.int32).reshape(NW, NCHUNK, C)
    zeros = jnp.zeros((N, D), jnp.float32)

    _segsum = _get_segsum()
    agg1 = _segsum(h, src, dst, zeros)
    l1, st = _mlp_stats(h, agg1[0], agg1[1],
                        W1a, b1a.reshape(1, D), W1b, b1b.reshape(1, D))
    mu = st[0] / N
    msq = st[1, 0] / N
    inv = lax.rsqrt(1e-6 + msq - jnp.sum(mu * mu))
    l1n = _norm_relu(l1, (mu * inv).reshape(1, D),
                     jnp.full((1, D), inv, jnp.float32))

    agg2 = _segsum(l1n, src, dst, zeros)
    l5 = _mlp(l1n, agg2[0], agg2[1],
              W4a, b4a.reshape(1, D), W4b, b4b.reshape(1, D))
    return (l5, l1n)


# speedup vs baseline: 9.6485x; 1.0247x over previous
import functools

import jax
import jax.numpy as jnp
from jax import lax
from jax.experimental import pallas as pl
from jax.experimental.pallas import tpu as pltpu
from jax.experimental.pallas import tpu_sc as plsc

N = 10000
E = 320000
D = 128

NC = 2
NS = 16
NW = NC * NS
EPT = E // NW
C = 80
NCHUNK = EPT // C
RPT = 624
TAIL = N - NS * RPT


def _segsum_body(x_hbm, src_hbm, dst_hbm, z_hbm, out_hbm,
                 sidx, didx, rows_a, rows_b, acc, sem_i, sem_a, sem_b):
    c = lax.axis_index("c")
    s = lax.axis_index("s")
    wid = c * NS + s
    rbase = s * RPT

    ci = pltpu.async_copy(src_hbm.at[wid], sidx, sem_i)
    cd = pltpu.async_copy(dst_hbm.at[wid], didx, sem_i)
    pltpu.sync_copy(z_hbm.at[pl.ds(rbase, RPT)], acc.at[pl.ds(rbase, RPT)])

    @pl.when(s == NS - 1)
    def _():
        pltpu.sync_copy(z_hbm.at[pl.ds(NS * RPT, TAIL)],
                        acc.at[pl.ds(NS * RPT, TAIL)])

    ci.wait()
    cd.wait()
    plsc.subcore_barrier()

    def gather(i, rbuf, sem):
        pltpu.async_copy(x_hbm.at[sidx.at[pl.ds(i * C, C)]], rbuf, sem)

    def gwait(rbuf, sem):
        pltpu.make_async_copy(x_hbm.at[sidx.at[pl.ds(0, C)]], rbuf, sem).wait()

    def scat(i, rbuf):
        pltpu.sync_copy(rbuf, acc.at[didx.at[i]], add=True)

    gather(0, rows_a, sem_a)

    def pair(b, carry):
        i = 2 * b
        gather(i + 1, rows_b, sem_b)
        gwait(rows_a, sem_a)
        scat(i, rows_a)

        gather(i + 2, rows_a, sem_a)
        gwait(rows_b, sem_b)
        scat(i + 1, rows_b)
        return carry

    lax.fori_loop(0, NCHUNK // 2, pair, 0)
    gwait(rows_a, sem_a)
    scat(NCHUNK - 1, rows_a)
    plsc.subcore_barrier()
    pltpu.sync_copy(acc.at[pl.ds(rbase, RPT)],
                    out_hbm.at[c].at[pl.ds(rbase, RPT)])

    @pl.when(s == NS - 1)
    def _():
        pltpu.sync_copy(acc.at[pl.ds(NS * RPT, TAIL)],
                        out_hbm.at[c].at[pl.ds(NS * RPT, TAIL)])


@functools.cache
def _get_segsum():
    return pl.kernel(
        _segsum_body,
        out_type=jax.ShapeDtypeStruct((NC, N, D), jnp.float32),
        mesh=plsc.VectorSubcoreMesh(core_axis_name="c", subcore_axis_name="s",
                                    num_cores=NC, num_subcores=NS),
        scratch_types=[
            pltpu.VMEM((EPT,), jnp.int32),
            pltpu.VMEM((NCHUNK, C), jnp.int32),
            pltpu.VMEM((C, D), jnp.float32),
            pltpu.VMEM((C, D), jnp.float32),
            pltpu.VMEM_SHARED((N, D), jnp.float32),
            pltpu.SemaphoreType.DMA,
            pltpu.SemaphoreType.DMA,
            pltpu.SemaphoreType.DMA,
        ],
    )


BLK = 1000


def _mm(x, w):
    return lax.dot_general(x, w, (((1,), (0,)), ((), ())),
                           preferred_element_type=jnp.float32,
                           precision=lax.Precision.HIGHEST)


NB = N // BLK


def _mlp_norm_body(h_ref, a0_ref, a1_ref, wa_ref, ba_ref, wb_ref, bb_ref,
                   out_ref, l1_scr, st_ref):
    i = pl.program_id(0)

    @pl.when(i < NB)
    def _():
        u = h_ref[...] + a0_ref[...] + a1_ref[...]
        t = jnp.maximum(_mm(u, wa_ref[...]) + ba_ref[...], 0.0)
        l1 = _mm(t, wb_ref[...]) + bb_ref[...]
        l1_scr[pl.ds(i * BLK, BLK), :] = l1

        @pl.when(i == 0)
        def _():
            st_ref[...] = jnp.zeros_like(st_ref)

        colsum = jnp.sum(l1, axis=0, keepdims=True)
        sumsq = jnp.broadcast_to(jnp.sum(l1 * l1), (1, D))
        st_ref[...] += jnp.concatenate(
            [colsum, sumsq, jnp.zeros((6, D), jnp.float32)], axis=0)

    @pl.when(i >= NB)
    def _():
        j = i - NB
        mu = st_ref[0:1, :] / N
        msq = st_ref[1, 0] / N
        inv = lax.rsqrt(1e-6 + msq - jnp.sum(mu * mu))
        blk = l1_scr[pl.ds(j * BLK, BLK), :]
        out_ref[...] = jnp.maximum((blk - mu) * inv, 0.0)


_mlp_norm = pl.pallas_call(
    _mlp_norm_body,
    grid=(2 * NB,),
    in_specs=[
        pl.BlockSpec((BLK, D), lambda i: (jnp.where(i < NB, i, NB - 1), 0)),
        pl.BlockSpec((BLK, D), lambda i: (jnp.where(i < NB, i, NB - 1), 0)),
        pl.BlockSpec((BLK, D), lambda i: (jnp.where(i < NB, i, NB - 1), 0)),
        pl.BlockSpec((D, D), lambda i: (0, 0)),
        pl.BlockSpec((1, D), lambda i: (0, 0)),
        pl.BlockSpec((D, D), lambda i: (0, 0)),
        pl.BlockSpec((1, D), lambda i: (0, 0)),
    ],
    out_specs=pl.BlockSpec((BLK, D), lambda i: (jnp.where(i < NB, 0, i - NB), 0)),
    out_shape=jax.ShapeDtypeStruct((N, D), jnp.float32),
    scratch_shapes=[
        pltpu.VMEM((N, D), jnp.float32),
        pltpu.VMEM((8, D), jnp.float32),
    ],
)


def _mlp_body(h_ref, a0_ref, a1_ref, wa_ref, ba_ref, wb_ref, bb_ref, out_ref):
    u = h_ref[...] + a0_ref[...] + a1_ref[...]
    t = jnp.maximum(_mm(u, wa_ref[...]) + ba_ref[...], 0.0)
    out_ref[...] = _mm(t, wb_ref[...]) + bb_ref[...]


_mlp = pl.pallas_call(
    _mlp_body,
    grid=(N // BLK,),
    in_specs=[
        pl.BlockSpec((BLK, D), lambda i: (i, 0)),
        pl.BlockSpec((BLK, D), lambda i: (i, 0)),
        pl.BlockSpec((BLK, D), lambda i: (i, 0)),
        pl.BlockSpec((D, D), lambda i: (0, 0)),
        pl.BlockSpec((1, D), lambda i: (0, 0)),
        pl.BlockSpec((D, D), lambda i: (0, 0)),
        pl.BlockSpec((1, D), lambda i: (0, 0)),
    ],
    out_specs=pl.BlockSpec((BLK, D), lambda i: (i, 0)),
    out_shape=jax.ShapeDtypeStruct((N, D), jnp.float32),
)


def kernel(h, edge_index, W1a, b1a, W1b, b1b, W4a, b4a, W4b, b4b):
    src = edge_index[0].astype(jnp.int32).reshape(NW, EPT)
    dst = edge_index[1].astype(jnp.int32).reshape(NW, NCHUNK, C)
    zeros = jnp.zeros((N, D), jnp.float32)

    _segsum = _get_segsum()
    agg1 = _segsum(h, src, dst, zeros)
    l1n = _mlp_norm(h, agg1[0], agg1[1],
                    W1a, b1a.reshape(1, D), W1b, b1b.reshape(1, D))

    agg2 = _segsum(l1n, src, dst, zeros)
    l5 = _mlp(l1n, agg2[0], agg2[1],
              W4a, b4a.reshape(1, D), W4b, b4b.reshape(1, D))
    return (l5, l1n)
